# baseline (speedup 1.0000x reference)
import functools

import jax
import jax.numpy as jnp
from jax import lax
from jax.experimental import pallas as pl
from jax.experimental.pallas import tpu as pltpu
from jax.experimental.pallas import tpu_sc as plsc

B = 16 * 128 * 128
D = 128
V = 500
NC, NS = 2, 16
NW = NC * NS
BPW = B // NW
C = 128
G = BPW // C
NBUF = 2
HC = C // 2


def _sc_body(x0, x1, x2, t0, t1, t2, out, idxp_v, idx_v, rows_v,
             sh0, sh1, sh2, gsem, osem):
    sid = lax.axis_index("s")
    wid = sid * NC + lax.axis_index("c")
    base = wid * BPW
    xs = (x0, x1, x2)
    tables = (sh0, sh1, sh2)

    @pl.when(sid == 0)
    def _():
        pltpu.sync_copy(t0, sh0)
        pltpu.sync_copy(t1, sh1)
        pltpu.sync_copy(t2, sh2)

    for t in range(3):
        pltpu.sync_copy(xs[t].at[pl.ds(wid * (G // 2), G // 2)],
                        idxp_v.at[t])
    plsc.subcore_barrier()

    def unpack_idx(g, b):
        row = lax.shift_right_logical(g, 1)
        half = lax.bitwise_and(g, 1) * HC
        lomask = jnp.full((16,), jnp.int32(0xFFFF))
        for t in range(3):
            for q in range(HC // 16):
                w = idxp_v[t, row, pl.ds(half + q * 16, 16)]
                idx_v[b, t, pl.ds(q * 16, 16)] = lax.bitwise_and(w, lomask)
                idx_v[b, t, pl.ds(64 + q * 16, 16)] = (
                    lax.shift_right_logical(w, 16))

    def issue_gathers(g, b):
        for t in range(3):
            pltpu.async_copy(tables[t].at[idx_v.at[b, t]], rows_v.at[b, t],
                             gsem.at[b])

    def wait_gathers(b):
        for t in range(3):
            pltpu.make_async_copy(tables[t].at[idx_v.at[b, t]],
                                  rows_v.at[b, t], gsem.at[b]).wait()

    def wait_out(b):
        pltpu.make_async_copy(rows_v.at[b, 0], out.at[pl.ds(base, C)],
                              osem.at[b]).wait()

    unpack_idx(0, 0)
    issue_gathers(0, 0)

    def step(g, b):
        nb = 1 - b
        wait_gathers(b)

        @pl.when(jnp.logical_and(g >= 1, g + 1 < G))
        def _():
            wait_out(nb)

        @pl.when(g + 1 < G)
        def _():
            unpack_idx(g + 1, nb)
            issue_gathers(g + 1, nb)

        def add_row(r, carry):
            for l in range(D // 16):
                s = pl.ds(l * 16, 16)
                plsc.addupdate(rows_v.at[b, 0, r, s],
                               rows_v[b, 1, r, s] + rows_v[b, 2, r, s])
            return carry

        lax.fori_loop(0, C, add_row, 0, unroll=4)
        pltpu.async_copy(rows_v.at[b, 0], out.at[pl.ds(base + g * C, C)],
                         osem.at[b])

    def outer(gg, carry):
        step(NBUF * gg, 0)
        step(NBUF * gg + 1, 1)
        return carry

    lax.fori_loop(0, G // NBUF, outer, 0)
    wait_out(0)
    wait_out(1)


@functools.partial(jax.jit, static_argnames=())
def _sc_lookup(x0, x1, x2, emb0, emb1, emb2):
    f = pl.kernel(
        _sc_body,
        out_type=jax.ShapeDtypeStruct((B, D), jnp.float32),
        mesh=plsc.VectorSubcoreMesh(core_axis_name="c", subcore_axis_name="s",
                                    num_cores=NC, num_subcores=NS),
        scratch_types=[
            pltpu.VMEM((3, G // 2, C), jnp.int32),
            pltpu.VMEM((NBUF, 3, C), jnp.int32),
            pltpu.VMEM((NBUF, 3, C, D), jnp.float32),
            pltpu.VMEM_SHARED((V, D), jnp.float32),
            pltpu.VMEM_SHARED((V, D), jnp.float32),
            pltpu.VMEM_SHARED((V, D), jnp.float32),
            pltpu.SemaphoreType.DMA((NBUF,)),
            pltpu.SemaphoreType.DMA((NBUF,)),
        ],
    )
    return f(x0, x1, x2, emb0, emb1, emb2)


def _pack_idx(xt):
    c = xt.reshape(B // C, C)
    p = c[:, :HC] | (c[:, HC:] << 16)
    return p.reshape(B // (2 * C), C)


def kernel(x, emb0, emb1, emb2):
    xf = x.reshape(B, 3)
    x0 = _pack_idx(xf[:, 0])
    x1 = _pack_idx(xf[:, 1])
    x2 = _pack_idx(xf[:, 2])
    out = _sc_lookup(x0, x1, x2, emb0, emb1, emb2)
    return out.reshape(x.shape[0], x.shape[1], x.shape[2], D)
